# baseline (speedup 1.0000x reference)
import jax
import jax.numpy as jnp
from jax.experimental import pallas as pl
from jax.experimental.pallas import tpu as pltpu

_NA = 128
_AS = 128
_NS = 2048
_B = 8
_STEPS = 25
_W = 2 * _AS
_ROW = _AS * _NA
_QP = _AS + _NS + _AS
_FLAT = _QP * _NA
_CEN0 = _AS * _NA
_CEN1 = _CEN0 + _NS * _NA
_SLOTS = 32
_CH = 2048
_NCH = _ROW // _CH
_BIG = 1 << 30

_HI = jax.lax.Precision.HIGHEST


def _dot(a, b):
    return jax.lax.dot_general(a, b, (((1,), (0,)), ((), ())),
                               precision=_HI,
                               preferred_element_type=jnp.float32)


def _slog(x):
    safe = jnp.where(x > 0, x, 1.0)
    return jnp.maximum(jnp.where(x > 0, jnp.log(safe), -100.0), -100.0)


def _mp_kernel(sigs_ref, d_ref, out_ref,
               d2f, f_s, c_s, r_s, dn_s, fi_s, vals_s, idx_s, sp_s,
               cm_s, ci_s):
    f32 = jnp.float32
    i32 = jnp.int32

    d = d_ref[...]
    nrm = jnp.sqrt(jnp.sum(d * d, axis=1, keepdims=True))
    dn = d / jnp.maximum(nrm, 1e-12)
    dn_s[...] = dn

    dT = jnp.swapaxes(dn, 0, 1)
    dTpad = jnp.concatenate([dT, jnp.zeros((_AS, _NA), f32)], axis=0)

    def _build_d2f(i, c):
        d2f[:, pl.ds(i * _NA, _NA)] = pltpu.roll(dTpad, i, axis=0)
        return c

    jax.lax.fori_loop(0, _AS, _build_d2f, 0)

    wd = jnp.concatenate([dn, jnp.zeros((_NA, _AS), f32)], axis=1)

    def _build_c(k, c):
        c_s[:, pl.ds(k * _CH, _CH)] = _dot(wd, d2f[:, pl.ds(k * _CH, _CH)])
        return c

    jax.lax.fori_loop(0, _NCH, _build_c, 0)

    r_s[:, 0:_NA] = jnp.zeros((_NA, _NA), f32)

    def _build_r(j, c):
        blk = c_s[:, pl.ds((_AS - j) * _NA, _NA)]
        r_s[:, pl.ds(j * _NA, _NA)] = jnp.swapaxes(blk, 0, 1)
        return c

    jax.lax.fori_loop(1, _AS, _build_r, 0)

    sp_s[...] = jnp.zeros((_B, _QP), f32)
    sp_s[:, 0:_NS] = sigs_ref[...]
    f_s[:, 0:_CEN0] = jnp.zeros((_B, _CEN0), f32)
    f_s[:, _CEN1:_FLAT] = jnp.zeros((_B, _FLAT - _CEN1), f32)

    def _init_f(n, c):
        j = n // _NCH
        k = n - j * _NCH
        w = sp_s[:, pl.ds(j * _AS, _W)]
        blk = _dot(w, d2f[:, pl.ds(k * _CH, _CH)])
        f_s[:, pl.ds(_CEN0 + j * _ROW + k * _CH, _CH)] = blk
        return c

    jax.lax.fori_loop(0, (_NS // _AS) * _NCH, _init_f, 0)

    li = jax.lax.broadcasted_iota(i32, (1, _NS * _NA), 1)
    fi_s[...] = (li % _NA) * _NS + li // _NA

    vals_s[...] = jnp.zeros((_SLOTS, _B), f32)
    idx_s[...] = jnp.full((_SLOTS, _B), -1, i32)

    def _stat0(t, c):
        ch = f_s[:, pl.ds(_CEN0 + t * _ROW, _ROW)]
        fich = fi_s[:, pl.ds(t * _ROW, _ROW)]
        chm = jnp.max(ch, axis=1, keepdims=True)
        chi = jnp.min(jnp.where(ch == chm, fich, _BIG), axis=1,
                      keepdims=True)
        cm_s[pl.ds(t, 1), :] = jnp.swapaxes(chm, 0, 1)
        ci_s[pl.ds(t, 1), :] = jnp.swapaxes(chi, 0, 1)
        return c

    jax.lax.fori_loop(0, _NS // _AS, _stat0, 0)

    bcol = jax.lax.broadcasted_iota(i32, (_B, 1), 0)
    ti16 = jax.lax.broadcasted_iota(i32, (_NS // _AS, _B), 0)
    bi16 = jax.lax.broadcasted_iota(i32, (_NS // _AS, _B), 1)

    def _step(step, carry):
        cm = cm_s[...]
        mrow = jnp.max(cm, axis=0, keepdims=True)
        irow = jnp.min(jnp.where(cm == mrow, ci_s[...], _BIG), axis=0,
                       keepdims=True)
        m = jnp.swapaxes(mrow, 0, 1)
        idxv = jnp.swapaxes(irow, 0, 1)

        vals_s[pl.ds(step, 1), :] = jnp.swapaxes(m, 0, 1)
        idx_s[pl.ds(step, 1), :] = jnp.swapaxes(idxv, 0, 1)

        def _upd(b, c):
            mb = jnp.sum(jnp.where(bcol == b, m, 0.0))
            ib = jnp.sum(jnp.where(bcol == b, idxv, 0))
            asel = ib // _NS
            pp = ib - asel * _NS
            base = pp * _NA
            rowr = r_s[pl.ds(asel, 1), :]
            rowc = c_s[pl.ds(asel, 1), :]
            cur0 = f_s[pl.ds(b, 1), pl.ds(base, _ROW)]
            f_s[pl.ds(b, 1), pl.ds(base, _ROW)] = cur0 - mb * rowr
            cur1 = f_s[pl.ds(b, 1), pl.ds(base + _ROW, _ROW)]
            f_s[pl.ds(b, 1), pl.ds(base + _ROW, _ROW)] = cur1 - mb * rowc

            @pl.when(pp >= _NS - _AS + 1)
            def _():
                dpad = jnp.concatenate(
                    [dn_s[pl.ds(asel, 1), :], jnp.zeros((1, _AS), f32)],
                    axis=1)
                rolled = pltpu.roll(dpad, pp - (_NS - _W), axis=1)
                lmask = jax.lax.broadcasted_iota(i32, (1, _W), 1) >= _AS
                w1 = jnp.where(lmask, rolled, 0.0)

                def _corr(k, c2):
                    lo = _CEN1 - _ROW + k * _CH
                    blk = _dot(w1, d2f[:, pl.ds(k * _CH, _CH)])
                    cur2 = f_s[pl.ds(b, 1), pl.ds(lo, _CH)]
                    f_s[pl.ds(b, 1), pl.ds(lo, _CH)] = cur2 + mb * blk
                    return c2

                jax.lax.fori_loop(0, _NCH, _corr, 0)

            tlo = jnp.maximum(pp // _AS - 1, 0)
            thi = jnp.minimum((pp + _AS - 1) // _AS, _NS // _AS - 1)
            for dt in range(3):
                t = tlo + dt

                @pl.when(t <= thi)
                def _():
                    ch = f_s[pl.ds(b, 1), pl.ds(_CEN0 + t * _ROW, _ROW)]
                    fich = fi_s[:, pl.ds(t * _ROW, _ROW)]
                    chm = jnp.max(ch)
                    chi = jnp.min(jnp.where(ch == chm, fich, _BIG))
                    sel = (ti16 == t) & (bi16 == b)
                    cm_s[...] = jnp.where(sel, chm, cm_s[...])
                    ci_s[...] = jnp.where(sel, chi, ci_s[...])

            return c

        jax.lax.fori_loop(0, _B, _upd, 0)
        return carry

    jax.lax.fori_loop(0, _STEPS, _step, 0)

    live_row = jax.lax.broadcasted_iota(i32, (1, _SLOTS), 1) < _STEPS
    live_col = jnp.swapaxes(live_row, 0, 1)
    jrow = jax.lax.broadcasted_iota(i32, (_SLOTS, _SLOTS), 1)
    icol = jax.lax.broadcasted_iota(i32, (_SLOTS, 1), 0)

    firsts, svals, idxcols = [], [], []
    for b in range(_B):
        idx_c = idx_s[:, b:b + 1]
        val_c = vals_s[:, b:b + 1]
        idx_b = jnp.swapaxes(idx_c, 0, 1)
        val_b = jnp.swapaxes(val_c, 0, 1)
        eq = (idx_c == idx_b) & live_col & live_row
        sval = jnp.sum(jnp.where(eq, val_b, 0.0), axis=1, keepdims=True)
        minj = jnp.min(jnp.where(eq, jrow, _SLOTS * 4), axis=1,
                       keepdims=True)
        firsts.append(minj == icol)
        svals.append(sval)
        idxcols.append(idx_c)

    mx = 0.0
    for b in range(_B):
        mx = jnp.maximum(mx, jnp.max(jnp.where(firsts[b], svals[b], -1e30)))

    total = 0.0
    for br in range(_B // 2):
        bt = br + _B // 2
        rv = jnp.where(firsts[br], svals[br] / mx, 0.0)
        tv_c = jnp.where(firsts[bt], svals[bt] / mx, 0.0)
        tv_r = jnp.swapaxes(tv_c, 0, 1)
        ft_r = jnp.swapaxes(firsts[bt], 0, 1)
        it_r = jnp.swapaxes(idxcols[bt], 0, 1)
        mm = firsts[br] & ft_r & (idxcols[br] == it_r)
        t_at_r = jnp.sum(jnp.where(mm, tv_r, 0.0), axis=1, keepdims=True)
        t_hit = jnp.max(jnp.where(mm, 1, 0), axis=0, keepdims=True)
        term_r = jnp.where(
            firsts[br],
            -(t_at_r * _slog(rv) + (1.0 - t_at_r) * _slog(1.0 - rv)),
            0.0)
        term_t = jnp.where(ft_r & (t_hit == 0), 100.0 * tv_r, 0.0)
        total = total + jnp.sum(term_r) + jnp.sum(term_t)

    loss = total * (1.0 / (4 * _NA * _NS))
    out_ref[...] = jnp.reshape(loss, (1, 1))


def kernel(recon, target, d):
    sigs = jnp.concatenate([recon, target], axis=0).astype(jnp.float32)
    out = pl.pallas_call(
        _mp_kernel,
        out_shape=jax.ShapeDtypeStruct((1, 1), jnp.float32),
        scratch_shapes=[
            pltpu.VMEM((_W, _ROW), jnp.float32),
            pltpu.VMEM((_B, _FLAT), jnp.float32),
            pltpu.VMEM((_NA, _ROW), jnp.float32),
            pltpu.VMEM((_NA, _ROW), jnp.float32),
            pltpu.VMEM((_NA, _AS), jnp.float32),
            pltpu.VMEM((1, _NS * _NA), jnp.int32),
            pltpu.VMEM((_SLOTS, _B), jnp.float32),
            pltpu.VMEM((_SLOTS, _B), jnp.int32),
            pltpu.VMEM((_B, _QP), jnp.float32),
            pltpu.VMEM((_NS // _AS, _B), jnp.float32),
            pltpu.VMEM((_NS // _AS, _B), jnp.int32),
        ],
    )(sigs, d.astype(jnp.float32))
    return out.reshape(())


# speedup vs baseline: 1.2544x; 1.2544x over previous
import jax
import jax.numpy as jnp
from jax.experimental import pallas as pl
from jax.experimental.pallas import tpu as pltpu

_NA = 128
_AS = 128
_NS = 2048
_B = 8
_STEPS = 25
_W = 2 * _AS
_ROW = _AS * _NA
_QP = _AS + _NS + _AS
_FLAT = _QP * _NA
_CEN0 = _AS * _NA
_CEN1 = _CEN0 + _NS * _NA
_SLOTS = 32
_CH = 2048
_NCH = _ROW // _CH
_BIG = 1 << 30

_HI = jax.lax.Precision.HIGHEST


def _dot(a, b):
    return jax.lax.dot_general(a, b, (((1,), (0,)), ((), ())),
                               precision=_HI,
                               preferred_element_type=jnp.float32)


def _slog(x):
    safe = jnp.where(x > 0, x, 1.0)
    return jnp.maximum(jnp.where(x > 0, jnp.log(safe), -100.0), -100.0)


def _mp_kernel(sigs_ref, d_ref, out_ref,
               d2f, f_s, c_s, r_s, dn_s, fi_s, vals_s, idx_s, sp_s):
    f32 = jnp.float32
    i32 = jnp.int32

    d = d_ref[...]
    nrm = jnp.sqrt(jnp.sum(d * d, axis=1, keepdims=True))
    dn = d / jnp.maximum(nrm, 1e-12)
    dn_s[...] = dn

    dT = jnp.swapaxes(dn, 0, 1)
    dTpad = jnp.concatenate([dT, jnp.zeros((_AS, _NA), f32)], axis=0)

    def _build_d2f(i, c):
        d2f[:, pl.ds(i * _NA, _NA)] = pltpu.roll(dTpad, i, axis=0)
        return c

    jax.lax.fori_loop(0, _AS, _build_d2f, 0)

    wd = jnp.concatenate([dn, jnp.zeros((_NA, _AS), f32)], axis=1)

    def _build_c(k, c):
        c_s[:, pl.ds(k * _CH, _CH)] = _dot(wd, d2f[:, pl.ds(k * _CH, _CH)])
        return c

    jax.lax.fori_loop(0, _NCH, _build_c, 0)

    r_s[:, 0:_NA] = jnp.zeros((_NA, _NA), f32)

    def _build_r(j, c):
        blk = c_s[:, pl.ds((_AS - j) * _NA, _NA)]
        r_s[:, pl.ds(j * _NA, _NA)] = jnp.swapaxes(blk, 0, 1)
        return c

    jax.lax.fori_loop(1, _AS, _build_r, 0)

    sp_s[...] = jnp.zeros((_B, _QP), f32)
    sp_s[:, 0:_NS] = sigs_ref[...]
    f_s[:, 0:_CEN0] = jnp.zeros((_B, _CEN0), f32)
    f_s[:, _CEN1:_FLAT] = jnp.zeros((_B, _FLAT - _CEN1), f32)

    def _init_f(n, c):
        j = n // _NCH
        k = n - j * _NCH
        w = sp_s[:, pl.ds(j * _AS, _W)]
        blk = _dot(w, d2f[:, pl.ds(k * _CH, _CH)])
        f_s[:, pl.ds(_CEN0 + j * _ROW + k * _CH, _CH)] = blk
        return c

    jax.lax.fori_loop(0, (_NS // _AS) * _NCH, _init_f, 0)

    li = jax.lax.broadcasted_iota(i32, (1, _NS * _NA), 1)
    fi_s[...] = (li % _NA) * _NS + li // _NA

    vals_s[...] = jnp.zeros((_SLOTS, _B), f32)
    idx_s[...] = jnp.full((_SLOTS, _B), -1, i32)

    bcol = jax.lax.broadcasted_iota(i32, (_B, 1), 0)

    def _step(step, carry):
        def _mx(k, st):
            rm, ri = st
            ch = f_s[:, pl.ds(_CEN0 + k * _ROW, _ROW)]
            fich = fi_s[:, pl.ds(k * _ROW, _ROW)]
            chm = jnp.max(ch, axis=1, keepdims=True)
            chi = jnp.min(jnp.where(ch == chm, fich, _BIG), axis=1,
                          keepdims=True)
            ri = jnp.where(chm > rm, chi,
                           jnp.where(chm == rm, jnp.minimum(ri, chi), ri))
            return jnp.maximum(rm, chm), ri

        m, idxv = jax.lax.fori_loop(
            0, _NS // _AS, _mx,
            (jnp.full((_B, 1), -1e30, f32), jnp.full((_B, 1), _BIG, i32)))

        vals_s[pl.ds(step, 1), :] = jnp.swapaxes(m, 0, 1)
        idx_s[pl.ds(step, 1), :] = jnp.swapaxes(idxv, 0, 1)

        def _upd(b, c):
            mb = jnp.sum(jnp.where(bcol == b, m, 0.0))
            ib = jnp.sum(jnp.where(bcol == b, idxv, 0))
            asel = ib // _NS
            pp = ib - asel * _NS
            base = pp * _NA
            rowr = r_s[pl.ds(asel, 1), :]
            rowc = c_s[pl.ds(asel, 1), :]
            cur0 = f_s[pl.ds(b, 1), pl.ds(base, _ROW)]
            f_s[pl.ds(b, 1), pl.ds(base, _ROW)] = cur0 - mb * rowr
            cur1 = f_s[pl.ds(b, 1), pl.ds(base + _ROW, _ROW)]
            f_s[pl.ds(b, 1), pl.ds(base + _ROW, _ROW)] = cur1 - mb * rowc

            @pl.when(pp >= _NS - _AS + 1)
            def _():
                dpad = jnp.concatenate(
                    [dn_s[pl.ds(asel, 1), :], jnp.zeros((1, _AS), f32)],
                    axis=1)
                rolled = pltpu.roll(dpad, pp - (_NS - _W), axis=1)
                lmask = jax.lax.broadcasted_iota(i32, (1, _W), 1) >= _AS
                w1 = jnp.where(lmask, rolled, 0.0)

                def _corr(k, c2):
                    lo = _CEN1 - _ROW + k * _CH
                    blk = _dot(w1, d2f[:, pl.ds(k * _CH, _CH)])
                    cur2 = f_s[pl.ds(b, 1), pl.ds(lo, _CH)]
                    f_s[pl.ds(b, 1), pl.ds(lo, _CH)] = cur2 + mb * blk
                    return c2

                jax.lax.fori_loop(0, _NCH, _corr, 0)

            return c

        jax.lax.fori_loop(0, _B, _upd, 0)
        return carry

    jax.lax.fori_loop(0, _STEPS, _step, 0)

    live_row = jax.lax.broadcasted_iota(i32, (1, _SLOTS), 1) < _STEPS
    live_col = jnp.swapaxes(live_row, 0, 1)
    jrow = jax.lax.broadcasted_iota(i32, (_SLOTS, _SLOTS), 1)
    icol = jax.lax.broadcasted_iota(i32, (_SLOTS, 1), 0)

    firsts, svals, idxcols = [], [], []
    for b in range(_B):
        idx_c = idx_s[:, b:b + 1]
        val_c = vals_s[:, b:b + 1]
        idx_b = jnp.swapaxes(idx_c, 0, 1)
        val_b = jnp.swapaxes(val_c, 0, 1)
        eq = (idx_c == idx_b) & live_col & live_row
        sval = jnp.sum(jnp.where(eq, val_b, 0.0), axis=1, keepdims=True)
        minj = jnp.min(jnp.where(eq, jrow, _SLOTS * 4), axis=1,
                       keepdims=True)
        firsts.append(minj == icol)
        svals.append(sval)
        idxcols.append(idx_c)

    mx = 0.0
    for b in range(_B):
        mx = jnp.maximum(mx, jnp.max(jnp.where(firsts[b], svals[b], -1e30)))

    total = 0.0
    for br in range(_B // 2):
        bt = br + _B // 2
        rv = jnp.where(firsts[br], svals[br] / mx, 0.0)
        tv_c = jnp.where(firsts[bt], svals[bt] / mx, 0.0)
        tv_r = jnp.swapaxes(tv_c, 0, 1)
        ft_r = jnp.swapaxes(firsts[bt], 0, 1)
        it_r = jnp.swapaxes(idxcols[bt], 0, 1)
        mm = firsts[br] & ft_r & (idxcols[br] == it_r)
        t_at_r = jnp.sum(jnp.where(mm, tv_r, 0.0), axis=1, keepdims=True)
        t_hit = jnp.max(jnp.where(mm, 1, 0), axis=0, keepdims=True)
        term_r = jnp.where(
            firsts[br],
            -(t_at_r * _slog(rv) + (1.0 - t_at_r) * _slog(1.0 - rv)),
            0.0)
        term_t = jnp.where(ft_r & (t_hit == 0), 100.0 * tv_r, 0.0)
        total = total + jnp.sum(term_r) + jnp.sum(term_t)

    loss = total * (1.0 / (4 * _NA * _NS))
    out_ref[...] = jnp.reshape(loss, (1, 1))


def kernel(recon, target, d):
    sigs = jnp.concatenate([recon, target], axis=0).astype(jnp.float32)
    out = pl.pallas_call(
        _mp_kernel,
        out_shape=jax.ShapeDtypeStruct((1, 1), jnp.float32),
        scratch_shapes=[
            pltpu.VMEM((_W, _ROW), jnp.float32),
            pltpu.VMEM((_B, _FLAT), jnp.float32),
            pltpu.VMEM((_NA, _ROW), jnp.float32),
            pltpu.VMEM((_NA, _ROW), jnp.float32),
            pltpu.VMEM((_NA, _AS), jnp.float32),
            pltpu.VMEM((1, _NS * _NA), jnp.int32),
            pltpu.VMEM((_SLOTS, _B), jnp.float32),
            pltpu.VMEM((_SLOTS, _B), jnp.int32),
            pltpu.VMEM((_B, _QP), jnp.float32),
        ],
    )(sigs, d.astype(jnp.float32))
    return out.reshape(())
